# baseline (speedup 1.0000x reference)
import functools

import jax
import jax.numpy as jnp
from jax import lax
from jax.experimental import pallas as pl
from jax.experimental.pallas import tpu as pltpu
from jax.experimental.pallas import tpu_sc as plsc

_BOUNDS = (-2.0, -1.5, -1.0, -0.5, 0.0, 0.5, 1.0, 1.5, 2.0)

_ROWS, _COLS = 16384, 26
_NC, _NS, _L = 2, 16, 16
_NW = _NC * _NS
_SPLIT = 4096
_TC_COLS = _ROWS - _SPLIT
_COLS_W = _SPLIT // _NW
_VECS = _COLS_W // _L

_mesh = plsc.VectorSubcoreMesh(core_axis_name="c", subcore_axis_name="s")


@functools.partial(
    pl.kernel,
    mesh=_mesh,
    out_type=jax.ShapeDtypeStruct((_COLS, _SPLIT), jnp.int32),
    scratch_types=[
        pltpu.VMEM((_COLS, _COLS_W), jnp.float32),
        pltpu.VMEM((_COLS, _COLS_W), jnp.int32),
    ],
    compiler_params=pltpu.CompilerParams(use_tc_tiling_on_sc=True),
)
def _bucketize_sc(x_hbm, out_hbm, x_v, o_v):
    wid = lax.axis_index("s") * _NC + lax.axis_index("c")
    c0 = wid * _COLS_W
    pltpu.sync_copy(x_hbm.at[:, pl.ds(c0, _COLS_W)], x_v)

    bvecs = [jnp.full((_L,), b, jnp.float32) for b in _BOUNDS]
    one = jnp.ones((_L,), jnp.int32)
    zero = jnp.zeros((_L,), jnp.int32)

    def bucketize(x):
        acc = zero
        for bv in bvecs:
            acc = acc + jnp.where(x > bv, one, zero)
        return acc

    @plsc.parallel_loop(0, _VECS, step=1)
    def body(v):
        base = v * _L
        for r in range(_COLS):
            o_v[r, pl.ds(base, _L)] = bucketize(x_v[r, pl.ds(base, _L)])

    pltpu.sync_copy(o_v, out_hbm.at[:, pl.ds(c0, _COLS_W)])


def _bucketize_tc_body(x_ref, o_ref):
    x = x_ref[...]
    acc = jnp.zeros(x.shape, jnp.int32)
    one = jnp.ones(x.shape, jnp.int32)
    for b in _BOUNDS:
        acc = acc + jnp.where(x > b, one, 0)
    o_ref[...] = acc


_TC_BLK = 4096
_bucketize_tc = pl.pallas_call(
    _bucketize_tc_body,
    out_shape=jax.ShapeDtypeStruct((_COLS, _TC_COLS), jnp.int32),
    grid=(_TC_COLS // _TC_BLK,),
    in_specs=[pl.BlockSpec((_COLS, _TC_BLK), lambda i: (0, i + _SPLIT // _TC_BLK))],
    out_specs=pl.BlockSpec((_COLS, _TC_BLK), lambda i: (0, i)),
)


def kernel(inputs):
    xt = inputs.T
    y_sc = _bucketize_sc(xt)
    y_tc = _bucketize_tc(xt)
    return jnp.concatenate([y_sc, y_tc], axis=1).T


# speedup vs baseline: 1.2040x; 1.2040x over previous
import functools

import jax
import jax.numpy as jnp
from jax import lax
from jax.experimental import pallas as pl
from jax.experimental.pallas import tpu as pltpu
from jax.experimental.pallas import tpu_sc as plsc

_BOUNDS = (-2.0, -1.5, -1.0, -0.5, 0.0, 0.5, 1.0, 1.5, 2.0)

_ROWS, _COLS = 16384, 26
_NC, _NS, _L = 2, 16, 16
_NW = _NC * _NS
_SC_ROWS = 8
_TC_ROWS = _COLS - _SC_ROWS
_SEG_W = _NW // _SC_ROWS
_SEG = _ROWS // _SEG_W
_VECS = _SEG // _L

_mesh = plsc.VectorSubcoreMesh(core_axis_name="c", subcore_axis_name="s")


@functools.partial(
    pl.kernel,
    mesh=_mesh,
    out_type=jax.ShapeDtypeStruct((_SC_ROWS, _ROWS), jnp.int32),
    scratch_types=[
        pltpu.VMEM((_SEG,), jnp.float32),
        pltpu.VMEM((_SEG,), jnp.int32),
    ],
    compiler_params=pltpu.CompilerParams(use_tc_tiling_on_sc=True),
)
def _bucketize_sc(x_hbm, out_hbm, x_v, o_v):
    wid = lax.axis_index("s") * _NC + lax.axis_index("c")
    row = wid // _SEG_W
    c0 = (wid % _SEG_W) * _SEG
    pltpu.sync_copy(x_hbm.at[row, pl.ds(c0, _SEG)], x_v)

    bvecs = [jnp.full((_L,), b, jnp.float32) for b in _BOUNDS]
    one = jnp.ones((_L,), jnp.int32)
    zero = jnp.zeros((_L,), jnp.int32)

    def bucketize(x):
        acc = zero
        for bv in bvecs:
            acc = acc + jnp.where(x > bv, one, zero)
        return acc

    @plsc.parallel_loop(0, _VECS, step=1, unroll=4)
    def body(v):
        base = v * _L
        o_v[pl.ds(base, _L)] = bucketize(x_v[pl.ds(base, _L)])

    pltpu.sync_copy(o_v, out_hbm.at[row, pl.ds(c0, _SEG)])


def _bucketize_tc_body(x_ref, o_ref):
    x = x_ref[...]
    acc = jnp.zeros(x.shape, jnp.int32)
    one = jnp.ones(x.shape, jnp.int32)
    for b in _BOUNDS:
        acc = acc + jnp.where(x > b, one, 0)
    o_ref[...] = acc


_TC_BLK = 8
_bucketize_tc = pl.pallas_call(
    _bucketize_tc_body,
    out_shape=jax.ShapeDtypeStruct((_TC_ROWS, _ROWS), jnp.int32),
    grid=((_TC_ROWS + _TC_BLK - 1) // _TC_BLK,),
    in_specs=[pl.BlockSpec((_TC_BLK, _ROWS), lambda i: (i + _SC_ROWS // _TC_BLK, 0))],
    out_specs=pl.BlockSpec((_TC_BLK, _ROWS), lambda i: (i, 0)),
)


def kernel(inputs):
    xt = inputs.T
    y_sc = _bucketize_sc(xt)
    y_tc = _bucketize_tc(xt)
    return jnp.concatenate([y_sc, y_tc], axis=0).T
